# Initial kernel scaffold; baseline (speedup 1.0000x reference)
import jax
import jax.numpy as jnp
from jax.experimental import pallas as pl


def kernel(queries, keys, values):
    raise NotImplementedError("write your pallas kernel here")


# speedup vs baseline: 30.7920x; 30.7920x over previous
import functools
import math

import jax
import jax.numpy as jnp
from jax.experimental import pallas as pl
from jax.experimental.pallas import tpu as pltpu

B, L, V, H, E = 2, 512, 16, 8, 32
D = H * E
TOP_K = int(math.log(L))
NEG = -1e30


def _roll_up_lanes(x, s):
    return jnp.concatenate([x[:, s:], x[:, :s]], axis=1)


def _corr_kernel(q_ref, k_ref, c_ref):
    for v in range(V):
        q = q_ref[0, :, v, :]
        k = k_ref[0, :, v, :]
        m = jax.lax.dot_general(
            k, q, (((1,), (1,)), ((), ())),
            preferred_element_type=jnp.float32,
            precision=jax.lax.Precision.HIGHEST,
        )
        h = L // 2
        while h >= 1:
            m = m[:h, :] + _roll_up_lanes(m[h:, :], h)
            h //= 2
        c_ref[0, v, :] = m[0, :] * (1.0 / D)


def _roll_right_lanes(x, s):
    n = x.shape[1]
    return jnp.concatenate([x[:, n - s:], x[:, :n - s]], axis=1)


def _agg_kernel(c_ref, v_ref, o_ref):
    bidx = pl.program_id(0)
    lane = jax.lax.broadcasted_iota(jnp.int32, (1, L), 1)
    prow = jax.lax.broadcasted_iota(jnp.int32, (D, D), 0)
    pcol = jax.lax.broadcasted_iota(jnp.int32, (D, D), 1)
    perm = (pcol == (prow % E) * H + prow // E).astype(jnp.float32)
    for v in range(V):
        c2 = c_ref[:, v, :]
        cm = (c2[0:1, :] + c2[1:2, :]) * 0.5
        rowmask = jax.lax.broadcasted_iota(jnp.int32, (B, L), 0) == bidx
        own = jnp.sum(jnp.where(rowmask, c2, 0.0), axis=0, keepdims=True)

        work = cm
        sel = jnp.zeros((1, L), jnp.bool_)
        for _ in range(TOP_K):
            mx = jnp.max(work)
            am = jnp.min(jnp.where(work == mx, lane, L))
            picked = lane == am
            sel = jnp.logical_or(sel, picked)
            work = jnp.where(picked, NEG, work)

        wv = jnp.where(sel, own, NEG)
        wmax = jnp.max(wv)
        ex = jnp.where(sel, jnp.exp(wv - wmax), 0.0)
        p = ex / jnp.sum(ex)

        cmat = p
        s = 1
        while s < L:
            cmat = jnp.concatenate([cmat, _roll_right_lanes(cmat, s)], axis=0)
            s *= 2
        vals = v_ref[0, :, v, :]
        agg = jax.lax.dot_general(
            cmat, vals, (((1,), (0,)), ((), ())),
            preferred_element_type=jnp.float32,
            precision=jax.lax.Precision.HIGHEST,
        )
        o_ref[0, :, v, :] = jax.lax.dot_general(
            agg, perm, (((1,), (0,)), ((), ())),
            preferred_element_type=jnp.float32,
            precision=jax.lax.Precision.HIGHEST,
        )


@jax.jit
def kernel(queries, keys, values):
    qr = queries.reshape(B, L, V, D)
    kr = keys.reshape(B, L, V, D)
    vr = values.reshape(B, L, V, D)

    c = pl.pallas_call(
        _corr_kernel,
        grid=(B,),
        in_specs=[
            pl.BlockSpec((1, L, V, D), lambda b: (b, 0, 0, 0)),
            pl.BlockSpec((1, L, V, D), lambda b: (b, 0, 0, 0)),
        ],
        out_specs=pl.BlockSpec((1, V, L), lambda b: (b, 0, 0)),
        out_shape=jax.ShapeDtypeStruct((B, V, L), jnp.float32),
    )(qr, kr)

    out = pl.pallas_call(
        _agg_kernel,
        grid=(B,),
        in_specs=[
            pl.BlockSpec((B, V, L), lambda b: (0, 0, 0)),
            pl.BlockSpec((1, L, V, D), lambda b: (b, 0, 0, 0)),
        ],
        out_specs=pl.BlockSpec((1, L, V, D), lambda b: (b, 0, 0, 0)),
        out_shape=jax.ShapeDtypeStruct((B, L, V, D), jnp.float32),
    )(c, vr)

    return out.reshape(B, L, V, E, H)


# speedup vs baseline: 48.9727x; 1.5904x over previous
import functools
import math

import jax
import jax.numpy as jnp
from jax.experimental import pallas as pl
from jax.experimental.pallas import tpu as pltpu

B, L, V, H, E = 2, 512, 16, 8, 32
D = H * E
TOP_K = int(math.log(L))
NEG = -1e30


def _roll_up_lanes(x, s):
    return jnp.concatenate([x[:, s:], x[:, :s]], axis=1)


def _corr_kernel(q_ref, k_ref, c_ref):
    for v in range(V):
        q = q_ref[0, :, v, :]
        k = k_ref[0, :, v, :]
        m = jax.lax.dot_general(
            k, q, (((1,), (1,)), ((), ())),
            preferred_element_type=jnp.float32,
            precision=jax.lax.Precision.HIGHEST,
        )
        h = L // 2
        while h >= 1:
            m = m[:h, :] + _roll_up_lanes(m[h:, :], h)
            h //= 2
        c_ref[0, v, :] = m[0, :] * (1.0 / D)


def _roll_right_lanes(x, s):
    n = x.shape[1]
    return jnp.concatenate([x[:, n - s:], x[:, :n - s]], axis=1)


def _select_weights(c_all, bidx):
    lane = jax.lax.broadcasted_iota(jnp.int32, (V, L), 1)
    cm = (c_all[0] + c_all[1]) * 0.5
    rowmask = jax.lax.broadcasted_iota(jnp.int32, (B, 1, 1), 0) == bidx
    own = jnp.sum(jnp.where(rowmask, c_all, 0.0), axis=0)

    work = cm
    sel = jnp.zeros((V, L), jnp.bool_)
    for _ in range(TOP_K):
        mx = jnp.max(work, axis=1, keepdims=True)
        am = jnp.min(jnp.where(work == mx, lane, L), axis=1, keepdims=True)
        picked = lane == am
        sel = jnp.logical_or(sel, picked)
        work = jnp.where(picked, NEG, work)

    wv = jnp.where(sel, own, NEG)
    wmax = jnp.max(wv, axis=1, keepdims=True)
    ex = jnp.where(sel, jnp.exp(wv - wmax), 0.0)
    return ex / jnp.sum(ex, axis=1, keepdims=True)


def _agg_kernel(c_ref, v_ref, o_ref):
    p_all = _select_weights(c_ref[...], pl.program_id(0))
    prow = jax.lax.broadcasted_iota(jnp.int32, (D, D), 0)
    pcol = jax.lax.broadcasted_iota(jnp.int32, (D, D), 1)
    perm = (pcol == (prow % E) * H + prow // E).astype(jnp.float32)
    for v in range(V):
        cmat = p_all[v:v + 1, :]
        s = 1
        while s < L:
            cmat = jnp.concatenate([cmat, _roll_right_lanes(cmat, s)], axis=0)
            s *= 2
        vals = v_ref[0, :, v, :]
        agg = jax.lax.dot_general(
            cmat, vals, (((1,), (0,)), ((), ())),
            preferred_element_type=jnp.float32,
            precision=jax.lax.Precision.DEFAULT,
        )
        o_ref[0, :, v, :] = jax.lax.dot_general(
            agg, perm, (((1,), (0,)), ((), ())),
            preferred_element_type=jnp.float32,
            precision=jax.lax.Precision.DEFAULT,
        )


@jax.jit
def kernel(queries, keys, values):
    qr = queries.reshape(B, L, V, D)
    kr = keys.reshape(B, L, V, D)
    vr = values.reshape(B, L, V, D)

    c = pl.pallas_call(
        _corr_kernel,
        grid=(B,),
        in_specs=[
            pl.BlockSpec((1, L, V, D), lambda b: (b, 0, 0, 0)),
            pl.BlockSpec((1, L, V, D), lambda b: (b, 0, 0, 0)),
        ],
        out_specs=pl.BlockSpec((1, V, L), lambda b: (b, 0, 0)),
        out_shape=jax.ShapeDtypeStruct((B, V, L), jnp.float32),
    )(qr, kr)

    out = pl.pallas_call(
        _agg_kernel,
        grid=(B,),
        in_specs=[
            pl.BlockSpec((B, V, L), lambda b: (0, 0, 0)),
            pl.BlockSpec((1, L, V, D), lambda b: (b, 0, 0, 0)),
        ],
        out_specs=pl.BlockSpec((1, L, V, D), lambda b: (b, 0, 0, 0)),
        out_shape=jax.ShapeDtypeStruct((B, L, V, D), jnp.float32),
    )(c, vr)

    return out.reshape(B, L, V, E, H)
